# baseline (speedup 1.0000x reference)
import functools

import jax
import jax.numpy as jnp
from jax import lax
from jax.experimental import pallas as pl
from jax.experimental.pallas import tpu as pltpu
from jax.experimental.pallas import tpu_sc as plsc

B, T, V, E, H = 4096, 20, 100000, 64, 64
BT = B * T
TP = T // 2

NC, NS = 2, 16
NW = NC * NS
PER_W = BT // NW
GROW = 128
MACRO = 640
GPM = MACRO // GROW
NMACRO = PER_W // MACRO
WSEG = 128
SPM = MACRO // WSEG


LSEG = 512
NLSEG = PER_W // LSEG


def _sc_gather(question, emb_table):
    mesh = plsc.VectorSubcoreMesh(core_axis_name="c", subcore_axis_name="s")

    @functools.partial(
        pl.kernel,
        mesh=mesh,
        out_type=jax.ShapeDtypeStruct((TP, B, 2 * E), jnp.float32),
        compiler_params=pltpu.CompilerParams(
            use_tc_tiling_on_sc=False, needs_layout_passes=False),
        scratch_types=[
            pltpu.VMEM((PER_W, T), jnp.int32),
            pltpu.VMEM((PER_W,), jnp.int32),
            pltpu.VMEM((MACRO, E), jnp.float32),
            pltpu.SemaphoreType.DMA,
        ],
    )
    def gather_kernel(idx_hbm, table_hbm, out_hbm, idx_v, idxc_v, rows_v, sem):
        wid = lax.axis_index("s") * NC + lax.axis_index("c")
        t_lo = wid // (NW // 2)
        base = (wid % (NW // 2)) * PER_W
        for p in range(NLSEG):
            b0 = (base + p * LSEG) % B
            pltpu.sync_copy(
                idx_hbm.at[pl.ds(b0, LSEG)],
                idx_v.at[pl.ds(p * LSEG, LSEG)],
            )
        lanes = lax.iota(jnp.int32, 16)
        for j in range(PER_W // 16):
            u = (base + j * 16) // B
            col = jnp.full((16,), 2 * u + t_lo, dtype=jnp.int32)
            vals = plsc.load_gather(idx_v, [j * 16 + lanes, col])
            idxc_v[pl.ds(j * 16, 16)] = vals
        for c in range(NMACRO):
            copies = []
            for j in range(GPM):
                g = c * GPM + j
                copies.append(
                    pltpu.async_copy(
                        table_hbm.at[idxc_v.at[pl.ds(g * GROW, GROW)]],
                        rows_v.at[pl.ds(j * GROW, GROW)],
                        sem,
                    )
                )
            for cp in copies:
                cp.wait()
            for s in range(SPM):
                r = base + c * MACRO + s * WSEG
                u = r // B
                b0 = r % B
                pltpu.sync_copy(
                    rows_v.at[pl.ds(s * WSEG, WSEG)],
                    out_hbm.at[u, pl.ds(b0, WSEG), pl.ds(t_lo * E, E)],
                )

    return gather_kernel(question, emb_table)


BBLK = 1024


def _gru_body(emb_ref, wpair_ref, whh_ref, bpair_ref, bhh_ref, out_ref):
    wpair = wpair_ref[...]
    whh = whh_ref[...]
    bpair = bpair_ref[...]
    bhh = bhh_ref[...]
    h = jnp.zeros((BBLK, H), dtype=jnp.float32)
    for u in range(TP):
        x2 = emb_ref[u]
        gi2 = jnp.dot(x2, wpair, preferred_element_type=jnp.float32) + bpair
        for t_lo in range(2):
            gi = gi2[:, t_lo * 3 * H:(t_lo + 1) * 3 * H]
            gh = jnp.dot(h, whh, preferred_element_type=jnp.float32) + bhh
            s = jax.nn.sigmoid(gi[:, :2 * H] + gh[:, :2 * H])
            r = s[:, :H]
            z = s[:, H:]
            n = jnp.tanh(gi[:, 2 * H:] + r * gh[:, 2 * H:])
            h = (1.0 - z) * n + z * h
    out_ref[...] = h


def _gru(emb3, wpair, whh, bpair, bhh):
    return pl.pallas_call(
        _gru_body,
        grid=(B // BBLK,),
        in_specs=[
            pl.BlockSpec((TP, BBLK, 2 * E), lambda i: (0, i, 0)),
            pl.BlockSpec((2 * E, 6 * H), lambda i: (0, 0)),
            pl.BlockSpec((H, 3 * H), lambda i: (0, 0)),
            pl.BlockSpec((1, 6 * H), lambda i: (0, 0)),
            pl.BlockSpec((1, 3 * H), lambda i: (0, 0)),
        ],
        out_specs=pl.BlockSpec((BBLK, H), lambda i: (i, 0)),
        out_shape=jax.ShapeDtypeStruct((B, H), jnp.float32),
    )(emb3, wpair, whh, bpair, bhh)


def kernel(question, question_lengths, pack_sequence, emb_table, W_ih, W_hh, b_ih, b_hh):
    emb3 = _sc_gather(question.astype(jnp.int32), emb_table)
    wihT = W_ih.T
    zeros = jnp.zeros_like(wihT)
    wpair = jnp.concatenate(
        [jnp.concatenate([wihT, zeros], axis=1),
         jnp.concatenate([zeros, wihT], axis=1)], axis=0)
    bpair = jnp.concatenate([b_ih, b_ih]).reshape(1, 6 * H)
    return _gru(emb3, wpair, W_hh.T, bpair, b_hh.reshape(1, 3 * H))


# speedup vs baseline: 1.0556x; 1.0556x over previous
import functools

import jax
import jax.numpy as jnp
from jax import lax
from jax.experimental import pallas as pl
from jax.experimental.pallas import tpu as pltpu
from jax.experimental.pallas import tpu_sc as plsc

B, T, V, E, H = 4096, 20, 100000, 64, 64
TP = T // 2

NCHUNK = 2
BC = B // NCHUNK
BTC = BC * T

NC, NS = 2, 16
NW = NC * NS
PER_W = BTC // NW
GROW = 128
MACRO = 640
GPM = MACRO // GROW
NMACRO = PER_W // MACRO
WSEG = 128
SPM = MACRO // WSEG
LSEG = 256
NLSEG = PER_W // LSEG


def _sc_gather(question, emb_table, ch):
    mesh = plsc.VectorSubcoreMesh(core_axis_name="c", subcore_axis_name="s")

    @functools.partial(
        pl.kernel,
        mesh=mesh,
        out_type=jax.ShapeDtypeStruct((TP, BC, 2 * E), jnp.float32),
        compiler_params=pltpu.CompilerParams(
            use_tc_tiling_on_sc=False, needs_layout_passes=False),
        scratch_types=[
            pltpu.VMEM((PER_W, T), jnp.int32),
            pltpu.VMEM((PER_W,), jnp.int32),
            pltpu.VMEM((MACRO, E), jnp.float32),
            pltpu.SemaphoreType.DMA,
        ],
    )
    def gather_kernel(idx_hbm, table_hbm, out_hbm, idx_v, idxc_v, rows_v, sem):
        wid = lax.axis_index("s") * NC + lax.axis_index("c")
        t_lo = wid // (NW // 2)
        base = (wid % (NW // 2)) * PER_W
        for p in range(NLSEG):
            b0 = (base + p * LSEG) % BC
            pltpu.sync_copy(
                idx_hbm.at[pl.ds(ch * BC + b0, LSEG)],
                idx_v.at[pl.ds(p * LSEG, LSEG)],
            )
        lanes = lax.iota(jnp.int32, 16)
        for j in range(PER_W // 16):
            u = (base + j * 16) // BC
            col = jnp.full((16,), 2 * u + t_lo, dtype=jnp.int32)
            idxc_v[pl.ds(j * 16, 16)] = plsc.load_gather(
                idx_v, [j * 16 + lanes, col])
        for c in range(NMACRO):
            copies = []
            for j in range(GPM):
                g = c * GPM + j
                copies.append(
                    pltpu.async_copy(
                        table_hbm.at[idxc_v.at[pl.ds(g * GROW, GROW)]],
                        rows_v.at[pl.ds(j * GROW, GROW)],
                        sem,
                    )
                )
            for cp in copies:
                cp.wait()
            for s in range(SPM):
                r = base + c * MACRO + s * WSEG
                u = r // BC
                b0 = r % BC
                pltpu.sync_copy(
                    rows_v.at[pl.ds(s * WSEG, WSEG)],
                    out_hbm.at[u, pl.ds(b0, WSEG), pl.ds(t_lo * E, E)],
                )

    return gather_kernel(question, emb_table)


BBLK = 1024


def _gru_body(emb_ref, wpair_ref, whh_ref, bpair_ref, bhh_ref, out_ref):
    wpair = wpair_ref[...]
    whh = whh_ref[...]
    bpair = bpair_ref[...]
    bhh = bhh_ref[...]
    h = jnp.zeros((BBLK, H), dtype=jnp.float32)
    for u in range(TP):
        x2 = emb_ref[u]
        gi2 = jnp.dot(x2, wpair, preferred_element_type=jnp.float32) + bpair
        for t_lo in range(2):
            gi = gi2[:, t_lo * 3 * H:(t_lo + 1) * 3 * H]
            gh = jnp.dot(h, whh, preferred_element_type=jnp.float32) + bhh
            s = jax.nn.sigmoid(gi[:, :2 * H] + gh[:, :2 * H])
            r = s[:, :H]
            z = s[:, H:]
            n = jnp.tanh(gi[:, 2 * H:] + r * gh[:, 2 * H:])
            h = (1.0 - z) * n + z * h
    out_ref[...] = h


def _gru(emb3, wpair, whh, bpair, bhh):
    return pl.pallas_call(
        _gru_body,
        grid=(BC // BBLK,),
        in_specs=[
            pl.BlockSpec((TP, BBLK, 2 * E), lambda i: (0, i, 0)),
            pl.BlockSpec((2 * E, 6 * H), lambda i: (0, 0)),
            pl.BlockSpec((H, 3 * H), lambda i: (0, 0)),
            pl.BlockSpec((1, 6 * H), lambda i: (0, 0)),
            pl.BlockSpec((1, 3 * H), lambda i: (0, 0)),
        ],
        out_specs=pl.BlockSpec((BBLK, H), lambda i: (i, 0)),
        out_shape=jax.ShapeDtypeStruct((BC, H), jnp.float32),
    )(emb3, wpair, whh, bpair, bhh)


def kernel(question, question_lengths, pack_sequence, emb_table, W_ih, W_hh, b_ih, b_hh):
    q = question.astype(jnp.int32)
    wihT = W_ih.T
    zeros = jnp.zeros_like(wihT)
    wpair = jnp.concatenate(
        [jnp.concatenate([wihT, zeros], axis=1),
         jnp.concatenate([zeros, wihT], axis=1)], axis=0)
    bpair = jnp.concatenate([b_ih, b_ih]).reshape(1, 6 * H)
    whhT = W_hh.T
    bhh2 = b_hh.reshape(1, 3 * H)
    outs = []
    for ch in range(NCHUNK):
        emb3 = _sc_gather(q, emb_table, ch)
        outs.append(_gru(emb3, wpair, whhT, bpair, bhh2))
    return jnp.concatenate(outs, axis=0)


# speedup vs baseline: 1.0654x; 1.0092x over previous
import functools

import jax
import jax.numpy as jnp
from jax import lax
from jax.experimental import pallas as pl
from jax.experimental.pallas import tpu as pltpu
from jax.experimental.pallas import tpu_sc as plsc

B, T, V, E, H = 4096, 20, 100000, 64, 64
TP = T // 2

NCHUNK = 2
BC = B // NCHUNK
BTC = BC * T

NC, NS = 2, 16
NW = NC * NS
PER_W = BTC // NW
GROW = 128
MACRO = 1280
GPM = MACRO // GROW
NMACRO = PER_W // MACRO
WSEG = 128
SPM = MACRO // WSEG
LSEG = 256
NLSEG = PER_W // LSEG


def _sc_gather(q1d, emb_table, ch):
    mesh = plsc.VectorSubcoreMesh(core_axis_name="c", subcore_axis_name="s")

    @functools.partial(
        pl.kernel,
        mesh=mesh,
        out_type=jax.ShapeDtypeStruct((TP, BC, 2 * E), jnp.float32),
        compiler_params=pltpu.CompilerParams(
            use_tc_tiling_on_sc=False, needs_layout_passes=False),
        scratch_types=[
            pltpu.VMEM((PER_W * T,), jnp.int32),
            pltpu.VMEM((PER_W,), jnp.int32),
            pltpu.VMEM((MACRO, E), jnp.float32),
            pltpu.SemaphoreType.DMA,
        ],
    )
    def gather_kernel(idx_hbm, table_hbm, out_hbm, idx_v, idxc_v, rows_v, sem):
        wid = lax.axis_index("s") * NC + lax.axis_index("c")
        t_lo = wid // (NW // 2)
        base = (wid % (NW // 2)) * PER_W
        for p in range(NLSEG):
            b0 = (base + p * LSEG) % BC
            pltpu.sync_copy(
                idx_hbm.at[pl.ds((ch * BC + b0) * T, LSEG * T)],
                idx_v.at[pl.ds(p * LSEG * T, LSEG * T)],
            )
        lanes = lax.iota(jnp.int32, 16)
        for j in range(PER_W // 16):
            u = (base + j * 16) // BC
            col = 2 * u + t_lo
            flat = (j * 16 + lanes) * T + col
            idxc_v[pl.ds(j * 16, 16)] = plsc.load_gather(idx_v, [flat])
        for c in range(NMACRO):
            copies = []
            for j in range(GPM):
                g = c * GPM + j
                copies.append(
                    pltpu.async_copy(
                        table_hbm.at[idxc_v.at[pl.ds(g * GROW, GROW)]],
                        rows_v.at[pl.ds(j * GROW, GROW)],
                        sem,
                    )
                )
            for cp in copies:
                cp.wait()
            for s in range(SPM):
                r = base + c * MACRO + s * WSEG
                u = r // BC
                b0 = r % BC
                pltpu.sync_copy(
                    rows_v.at[pl.ds(s * WSEG, WSEG)],
                    out_hbm.at[u, pl.ds(b0, WSEG), pl.ds(t_lo * E, E)],
                )

    return gather_kernel(q1d, emb_table)


BBLK = 1024


def _gru_body(emb_ref, wpair_ref, whh_ref, bpair_ref, bhh_ref, out_ref):
    wpair = wpair_ref[...]
    whh = whh_ref[...]
    bpair = bpair_ref[...]
    bhh = bhh_ref[...]
    h = jnp.zeros((BBLK, H), dtype=jnp.float32)
    for u in range(TP):
        x2 = emb_ref[u]
        gi2 = jnp.dot(x2, wpair, preferred_element_type=jnp.float32) + bpair
        for t_lo in range(2):
            gi = gi2[:, t_lo * 3 * H:(t_lo + 1) * 3 * H]
            gh = jnp.dot(h, whh, preferred_element_type=jnp.float32) + bhh
            s = jax.nn.sigmoid(gi[:, :2 * H] + gh[:, :2 * H])
            r = s[:, :H]
            z = s[:, H:]
            n = jnp.tanh(gi[:, 2 * H:] + r * gh[:, 2 * H:])
            h = (1.0 - z) * n + z * h
    out_ref[...] = h


def _gru(emb3, wpair, whh, bpair, bhh):
    return pl.pallas_call(
        _gru_body,
        grid=(BC // BBLK,),
        in_specs=[
            pl.BlockSpec((TP, BBLK, 2 * E), lambda i: (0, i, 0)),
            pl.BlockSpec((2 * E, 6 * H), lambda i: (0, 0)),
            pl.BlockSpec((H, 3 * H), lambda i: (0, 0)),
            pl.BlockSpec((1, 6 * H), lambda i: (0, 0)),
            pl.BlockSpec((1, 3 * H), lambda i: (0, 0)),
        ],
        out_specs=pl.BlockSpec((BBLK, H), lambda i: (i, 0)),
        out_shape=jax.ShapeDtypeStruct((BC, H), jnp.float32),
    )(emb3, wpair, whh, bpair, bhh)


def kernel(question, question_lengths, pack_sequence, emb_table, W_ih, W_hh, b_ih, b_hh):
    q1d = question.reshape(B * T).astype(jnp.int32)
    wihT = W_ih.T
    zeros = jnp.zeros_like(wihT)
    wpair = jnp.concatenate(
        [jnp.concatenate([wihT, zeros], axis=1),
         jnp.concatenate([zeros, wihT], axis=1)], axis=0)
    bpair = jnp.concatenate([b_ih, b_ih]).reshape(1, 6 * H)
    whhT = W_hh.T
    bhh2 = b_hh.reshape(1, 3 * H)
    outs = []
    for ch in range(NCHUNK):
        emb3 = _sc_gather(q1d, emb_table, ch)
        outs.append(_gru(emb3, wpair, whhT, bpair, bhh2))
    return jnp.concatenate(outs, axis=0)
